# baseline (speedup 1.0000x reference)
import jax
import jax.numpy as jnp
from jax import lax
from jax.experimental import pallas as pl
from jax.experimental.pallas import tpu as pltpu
from jax.experimental.pallas import tpu_sc as plsc

N_NODES = 10000
D = 128
E = 320000
NC = 2
NS = 16
NW = NC * NS
E_PER_W = E // NW
EC = 80
NCHUNK = E_PER_W // EC
NG = EC // 16


def _decoder_body(z_hbm, col_hbm, row_hbm, out_hbm,
                  cols, rows, zc0, zr0, zc1, zr1, outv, pscr, sem0, sem1):
    cid = lax.axis_index("c")
    sid = lax.axis_index("s")
    wid = sid * NC + cid
    base = wid * E_PER_W
    lane = lax.iota(jnp.int32, 16)
    lane17 = lane * 17

    pltpu.sync_copy(col_hbm.at[pl.ds(base, E_PER_W)], cols)
    pltpu.sync_copy(row_hbm.at[pl.ds(base, E_PER_W)], rows)

    def fetch(c, zc, zr, sem):
        off = c * EC
        h0 = pltpu.async_copy(z_hbm.at[cols.at[pl.ds(off, EC)]], zc, sem)
        h1 = pltpu.async_copy(z_hbm.at[rows.at[pl.ds(off, EC)]], zr, sem)
        return h0, h1

    def compute(c, zc, zr):
        off = base + c * EC

        def store_phase(g):
            e0 = g * 16
            pbase = (g & 1) * 272
            for j in range(16):
                e = e0 + j
                acc0 = zc[e, pl.ds(0, 16)] * zr[e, pl.ds(0, 16)]
                acc1 = zc[e, pl.ds(16, 16)] * zr[e, pl.ds(16, 16)]
                for k in range(2, D // 16, 2):
                    acc0 = acc0 + zc[e, pl.ds(k * 16, 16)] * zr[e, pl.ds(k * 16, 16)]
                    acc1 = acc1 + zc[e, pl.ds(k * 16 + 16, 16)] * zr[e, pl.ds(k * 16 + 16, 16)]
                pscr[pl.ds(pbase + j * 17, 16)] = acc0 + acc1

        def reduce_phase(g):
            pbase = (g & 1) * 272
            t = [plsc.load_gather(pscr, [lane17 + (pbase + l)]) for l in range(16)]
            while len(t) > 1:
                t = [t[i] + t[i + 1] for i in range(0, len(t), 2)]
            sig = 1.0 / (1.0 + jnp.exp(-t[0]))
            outv[pl.ds(g * 16, 16)] = sig

        store_phase(0)

        def group(g, carry2):
            store_phase(g)
            reduce_phase(g - 1)
            return carry2

        lax.fori_loop(1, NG, group, 0)
        reduce_phase(NG - 1)
        pltpu.sync_copy(outv, out_hbm.at[pl.ds(off, EC)])

    p0, p1 = fetch(0, zc0, zr0, sem0)
    p0.wait()
    p1.wait()

    def step(i, carry):
        c = i * 2
        h0, h1 = fetch(c + 1, zc1, zr1, sem1)
        compute(c, zc0, zr0)
        h0.wait()
        h1.wait()
        g0, g1 = fetch(c + 2, zc0, zr0, sem0)
        compute(c + 1, zc1, zr1)
        g0.wait()
        g1.wait()
        return carry

    lax.fori_loop(0, (NCHUNK - 1) // 2, step, 0)
    compute(NCHUNK - 1, zc0, zr0)


def kernel(z, edge_index):
    ei = edge_index.astype(jnp.int32)
    col = ei[0]
    row = ei[1]
    mesh = plsc.VectorSubcoreMesh(core_axis_name="c", subcore_axis_name="s")
    f = pl.kernel(
        _decoder_body,
        mesh=mesh,
        out_type=jax.ShapeDtypeStruct((E,), jnp.float32),
        compiler_params=pltpu.CompilerParams(needs_layout_passes=False),
        scratch_types=[
            pltpu.VMEM((E_PER_W,), jnp.int32),
            pltpu.VMEM((E_PER_W,), jnp.int32),
            pltpu.VMEM((EC, D), jnp.float32),
            pltpu.VMEM((EC, D), jnp.float32),
            pltpu.VMEM((EC, D), jnp.float32),
            pltpu.VMEM((EC, D), jnp.float32),
            pltpu.VMEM((EC,), jnp.float32),
            pltpu.VMEM((2 * 16 * 17,), jnp.float32),
            pltpu.SemaphoreType.DMA,
            pltpu.SemaphoreType.DMA,
        ],
    )
    return f(z, col, row)


# speedup vs baseline: 1.3642x; 1.3642x over previous
import jax
import jax.numpy as jnp
from jax import lax
from jax.experimental import pallas as pl
from jax.experimental.pallas import tpu as pltpu
from jax.experimental.pallas import tpu_sc as plsc

N_NODES = 10000
D = 128
E = 320000
NC = 2
NS = 16
NW = NC * NS
E_PER_W = E // NW
EC = 80
NCHUNK = E_PER_W // EC
NG = EC // 16


def _decoder_body(z_hbm, col_hbm, row_hbm, out_hbm,
                  cols, rows, zc0, zr0, zc1, zr1, outv, pscr, sem0, sem1):
    cid = lax.axis_index("c")
    sid = lax.axis_index("s")
    wid = sid * NC + cid
    base = wid * E_PER_W
    lane = lax.iota(jnp.int32, 16)
    lane17 = lane * 17

    pltpu.sync_copy(col_hbm.at[pl.ds(base, E_PER_W)], cols)
    pltpu.sync_copy(row_hbm.at[pl.ds(base, E_PER_W)], rows)

    def fetch(c, zc, zr, sem):
        off = c * EC
        h0 = pltpu.async_copy(z_hbm.at[cols.at[pl.ds(off, EC)]], zc, sem)
        h1 = pltpu.async_copy(z_hbm.at[rows.at[pl.ds(off, EC)]], zr, sem)
        return h0, h1

    def compute(c, zc, zr):
        off = base + c * EC

        def group(g, carry2):
            e0 = g * 16
            for j in range(16):
                e = e0 + j
                acc0 = zc[e, pl.ds(0, 16)] * zr[e, pl.ds(0, 16)]
                acc1 = zc[e, pl.ds(16, 16)] * zr[e, pl.ds(16, 16)]
                for k in range(2, D // 16, 2):
                    acc0 = acc0 + zc[e, pl.ds(k * 16, 16)] * zr[e, pl.ds(k * 16, 16)]
                    acc1 = acc1 + zc[e, pl.ds(k * 16 + 16, 16)] * zr[e, pl.ds(k * 16 + 16, 16)]
                pscr[pl.ds(j * 17, 16)] = acc0 + acc1
            t = [plsc.load_gather(pscr, [lane17 + l]) for l in range(16)]
            while len(t) > 1:
                t = [t[i] + t[i + 1] for i in range(0, len(t), 2)]
            sig = 1.0 / (1.0 + jnp.exp(-t[0]))
            outv[pl.ds(e0, 16)] = sig
            return carry2

        lax.fori_loop(0, NG, group, 0)
        pltpu.sync_copy(outv, out_hbm.at[pl.ds(off, EC)])

    p0, p1 = fetch(0, zc0, zr0, sem0)
    p0.wait()
    p1.wait()

    def step(i, carry):
        c = i * 2
        h0, h1 = fetch(c + 1, zc1, zr1, sem1)
        compute(c, zc0, zr0)
        h0.wait()
        h1.wait()
        g0, g1 = fetch(c + 2, zc0, zr0, sem0)
        compute(c + 1, zc1, zr1)
        g0.wait()
        g1.wait()
        return carry

    lax.fori_loop(0, (NCHUNK - 1) // 2, step, 0)
    compute(NCHUNK - 1, zc0, zr0)


def kernel(z, edge_index):
    ei = edge_index.astype(jnp.int32)
    col = ei[0]
    row = ei[1]
    mesh = plsc.VectorSubcoreMesh(core_axis_name="c", subcore_axis_name="s")
    f = pl.kernel(
        _decoder_body,
        mesh=mesh,
        out_type=jax.ShapeDtypeStruct((E,), jnp.float32),
        compiler_params=pltpu.CompilerParams(needs_layout_passes=False),
        scratch_types=[
            pltpu.VMEM((E_PER_W,), jnp.int32),
            pltpu.VMEM((E_PER_W,), jnp.int32),
            pltpu.VMEM((EC, D), jnp.float32),
            pltpu.VMEM((EC, D), jnp.float32),
            pltpu.VMEM((EC, D), jnp.float32),
            pltpu.VMEM((EC, D), jnp.float32),
            pltpu.VMEM((EC,), jnp.float32),
            pltpu.VMEM((16 * 17,), jnp.float32),
            pltpu.SemaphoreType.DMA,
            pltpu.SemaphoreType.DMA,
        ],
    )
    return f(z, col, row)


# speedup vs baseline: 1.3650x; 1.0006x over previous
import jax
import jax.numpy as jnp
from jax import lax
from jax.experimental import pallas as pl
from jax.experimental.pallas import tpu as pltpu
from jax.experimental.pallas import tpu_sc as plsc

N_NODES = 10000
D = 128
E = 320000
NC = 2
NS = 16
NW = NC * NS
E_PER_W = E // NW
EC = 80
NCHUNK = E_PER_W // EC
NG = EC // 16


def _decoder_body(z_hbm, col_hbm, row_hbm, out_hbm,
                  cols, rows, zc0, zr0, zc1, zr1, outv, pscr, sem0, sem1):
    cid = lax.axis_index("c")
    sid = lax.axis_index("s")
    wid = sid * NC + cid
    base = wid * E_PER_W
    lane = lax.iota(jnp.int32, 16)
    lane17 = lane * 17

    pltpu.sync_copy(col_hbm.at[pl.ds(base, E_PER_W)], cols)
    pltpu.sync_copy(row_hbm.at[pl.ds(base, E_PER_W)], rows)

    def fetch(c, zc, zr, sem):
        off = c * EC
        h0 = pltpu.async_copy(z_hbm.at[cols.at[pl.ds(off, EC)]], zc, sem)
        h1 = pltpu.async_copy(z_hbm.at[rows.at[pl.ds(off, EC)]], zr, sem)
        return h0, h1

    def compute(c, zc, zr):
        off = base + c * EC

        def load_edge(e):
            return ([zc[e, pl.ds(k * 16, 16)] for k in range(D // 16)],
                    [zr[e, pl.ds(k * 16, 16)] for k in range(D // 16)])

        def group(g, carry2):
            e0 = g * 16
            pa, pb = load_edge(e0)
            for j in range(16):
                if j < 15:
                    na, nb = load_edge(e0 + j + 1)
                m = [pa[k] * pb[k] for k in range(D // 16)]
                s0 = (m[0] + m[1]) + (m[2] + m[3])
                s1 = (m[4] + m[5]) + (m[6] + m[7])
                pscr[pl.ds(j * 17, 16)] = s0 + s1
                if j < 15:
                    pa, pb = na, nb
            t = [plsc.load_gather(pscr, [lane17 + l]) for l in range(16)]
            while len(t) > 1:
                t = [t[i] + t[i + 1] for i in range(0, len(t), 2)]
            sig = 1.0 / (1.0 + jnp.exp(-t[0]))
            outv[pl.ds(e0, 16)] = sig
            return carry2

        lax.fori_loop(0, NG, group, 0)
        pltpu.sync_copy(outv, out_hbm.at[pl.ds(off, EC)])

    p0, p1 = fetch(0, zc0, zr0, sem0)
    p0.wait()
    p1.wait()

    def step(i, carry):
        c = i * 2
        h0, h1 = fetch(c + 1, zc1, zr1, sem1)
        compute(c, zc0, zr0)
        h0.wait()
        h1.wait()
        g0, g1 = fetch(c + 2, zc0, zr0, sem0)
        compute(c + 1, zc1, zr1)
        g0.wait()
        g1.wait()
        return carry

    lax.fori_loop(0, (NCHUNK - 1) // 2, step, 0)
    compute(NCHUNK - 1, zc0, zr0)


def kernel(z, edge_index):
    ei = edge_index.astype(jnp.int32)
    col = ei[0]
    row = ei[1]
    mesh = plsc.VectorSubcoreMesh(core_axis_name="c", subcore_axis_name="s")
    f = pl.kernel(
        _decoder_body,
        mesh=mesh,
        out_type=jax.ShapeDtypeStruct((E,), jnp.float32),
        compiler_params=pltpu.CompilerParams(needs_layout_passes=False),
        scratch_types=[
            pltpu.VMEM((E_PER_W,), jnp.int32),
            pltpu.VMEM((E_PER_W,), jnp.int32),
            pltpu.VMEM((EC, D), jnp.float32),
            pltpu.VMEM((EC, D), jnp.float32),
            pltpu.VMEM((EC, D), jnp.float32),
            pltpu.VMEM((EC, D), jnp.float32),
            pltpu.VMEM((EC,), jnp.float32),
            pltpu.VMEM((16 * 17,), jnp.float32),
            pltpu.SemaphoreType.DMA,
            pltpu.SemaphoreType.DMA,
        ],
    )
    return f(z, col, row)


# speedup vs baseline: 1.9505x; 1.4289x over previous
import jax
import jax.numpy as jnp
from jax import lax
from jax.experimental import pallas as pl
from jax.experimental.pallas import tpu as pltpu
from jax.experimental.pallas import tpu_sc as plsc

N_NODES = 10000
D = 128
E = 320000
NC = 2
NS = 16
NW = NC * NS
E_PER_W = E // NW
EC = 80
NCHUNK = E_PER_W // EC
NG = EC // 16


def _decoder_body(z_hbm, col_hbm, row_hbm, out_hbm,
                  cols, rows, zc0, zr0, zc1, zr1, zc2, zr2, zc3, zr3,
                  outv, pscr, sem0, sem1, sem2, sem3):
    cid = lax.axis_index("c")
    sid = lax.axis_index("s")
    wid = sid * NC + cid
    base = wid * E_PER_W
    lane = lax.iota(jnp.int32, 16)
    lane17 = lane * 17

    pltpu.sync_copy(col_hbm.at[pl.ds(base, E_PER_W)], cols)
    pltpu.sync_copy(row_hbm.at[pl.ds(base, E_PER_W)], rows)

    def fetch(c, zc, zr, sem):
        off = jnp.minimum(c, NCHUNK - 1) * EC
        pltpu.async_copy(z_hbm.at[cols.at[pl.ds(off, EC)]], zc, sem)
        pltpu.async_copy(z_hbm.at[rows.at[pl.ds(off, EC)]], zr, sem)

    def drain(zc, zr, sem):
        pltpu.make_async_copy(z_hbm.at[pl.ds(0, EC)], zc, sem).wait()
        pltpu.make_async_copy(z_hbm.at[pl.ds(0, EC)], zr, sem).wait()

    def compute(c, zc, zr):
        off = base + c * EC

        def load_edge(e):
            return ([zc[e, pl.ds(k * 16, 16)] for k in range(D // 16)],
                    [zr[e, pl.ds(k * 16, 16)] for k in range(D // 16)])

        def group(g, carry2):
            e0 = g * 16
            pa, pb = load_edge(e0)
            for j in range(16):
                if j < 15:
                    na, nb = load_edge(e0 + j + 1)
                m = [pa[k] * pb[k] for k in range(D // 16)]
                s0 = (m[0] + m[1]) + (m[2] + m[3])
                s1 = (m[4] + m[5]) + (m[6] + m[7])
                pscr[pl.ds(j * 17, 16)] = s0 + s1
                if j < 15:
                    pa, pb = na, nb
            t = [plsc.load_gather(pscr, [lane17 + l]) for l in range(16)]
            while len(t) > 1:
                t = [t[i] + t[i + 1] for i in range(0, len(t), 2)]
            sig = 1.0 / (1.0 + jnp.exp(-t[0]))
            outv[pl.ds(e0, 16)] = sig
            return carry2

        lax.fori_loop(0, NG, group, 0)
        pltpu.sync_copy(outv, out_hbm.at[pl.ds(off, EC)])

    fetch(0, zc0, zr0, sem0)
    fetch(1, zc1, zr1, sem1)
    fetch(2, zc2, zr2, sem2)
    drain(zc0, zr0, sem0)

    def step(i, carry):
        c = i * 4
        fetch(c + 3, zc3, zr3, sem3)
        compute(c, zc0, zr0)
        drain(zc1, zr1, sem1)
        fetch(c + 4, zc0, zr0, sem0)
        compute(c + 1, zc1, zr1)
        drain(zc2, zr2, sem2)
        fetch(c + 5, zc1, zr1, sem1)
        compute(c + 2, zc2, zr2)
        drain(zc3, zr3, sem3)
        fetch(c + 6, zc2, zr2, sem2)
        compute(c + 3, zc3, zr3)
        drain(zc0, zr0, sem0)
        return carry

    lax.fori_loop(0, (NCHUNK - 1) // 4, step, 0)
    compute(NCHUNK - 1, zc0, zr0)
    drain(zc1, zr1, sem1)
    drain(zc2, zr2, sem2)


def kernel(z, edge_index):
    ei = edge_index.astype(jnp.int32)
    col = ei[0]
    row = ei[1]
    mesh = plsc.VectorSubcoreMesh(core_axis_name="c", subcore_axis_name="s")
    f = pl.kernel(
        _decoder_body,
        mesh=mesh,
        out_type=jax.ShapeDtypeStruct((E,), jnp.float32),
        compiler_params=pltpu.CompilerParams(needs_layout_passes=False),
        scratch_types=[
            pltpu.VMEM((E_PER_W,), jnp.int32),
            pltpu.VMEM((E_PER_W,), jnp.int32),
            pltpu.VMEM((EC, D), jnp.float32),
            pltpu.VMEM((EC, D), jnp.float32),
            pltpu.VMEM((EC, D), jnp.float32),
            pltpu.VMEM((EC, D), jnp.float32),
            pltpu.VMEM((EC, D), jnp.float32),
            pltpu.VMEM((EC, D), jnp.float32),
            pltpu.VMEM((EC, D), jnp.float32),
            pltpu.VMEM((EC, D), jnp.float32),
            pltpu.VMEM((EC,), jnp.float32),
            pltpu.VMEM((16 * 17,), jnp.float32),
            pltpu.SemaphoreType.DMA,
            pltpu.SemaphoreType.DMA,
            pltpu.SemaphoreType.DMA,
            pltpu.SemaphoreType.DMA,
        ],
    )
    return f(z, col, row)
